# baseline (speedup 1.0000x reference)
import functools

import jax
import jax.numpy as jnp
from jax import lax
from jax.experimental import pallas as pl
from jax.experimental.pallas import tpu as pltpu
from jax.experimental.pallas import tpu_sc as plsc

N_CODES = 1024
EMBED_DIM = 256
N_TOKENS = 16384
BLK = 4096
GRID = N_TOKENS // BLK
EPS = 1e-12


def _tc_body(z_ref, emb_ref, idx_ref, loss_ref, perp_ref,
             embn_ref, acc_ref, znsq_ref, kmat_ref):
    pid = pl.program_id(0)

    @pl.when(pid == 0)
    def _init():
        e = emb_ref[...]
        ss = jnp.sum(e * e, axis=1, keepdims=True)
        nc = jnp.maximum(jnp.sqrt(ss), EPS)
        embn_ref[...] = e / nc
        ssr = lax.dot_general(jnp.ones((1, EMBED_DIM), jnp.float32), e * e,
                              (((1,), (1,)), ((), ())),
                              preferred_element_type=jnp.float32,
                              precision=lax.Precision.HIGHEST)
        ncr = jnp.maximum(jnp.sqrt(ssr), EPS)
        acc_ref[...] = jnp.zeros((1, N_CODES), jnp.float32)
        znsq_ref[0, 0] = 0.0
        znsq_ref[0, 1] = 0.0
        znsq_ref[0, 2] = 0.0
        kv = lax.broadcasted_iota(jnp.int32, (5, N_CODES), 1)
        cv = lax.broadcasted_iota(jnp.int32, (5, N_CODES), 0)
        base = jnp.where(cv == 0, kv >> 5,
                         jnp.where(cv == 1, kv & 31, 1)).astype(jnp.float32)
        kmat_ref[...] = jnp.where(cv == 3, ncr,
                                  jnp.where(cv == 4, ssr, base)
                                  ).astype(jnp.bfloat16)

    z = z_ref[...]
    zn = z / jnp.maximum(jnp.sqrt(jnp.sum(z * z, axis=1, keepdims=True)), EPS)

    sim = lax.dot_general(zn, embn_ref[...], (((1,), (1,)), ((), ())),
                          preferred_element_type=jnp.float32)

    dist = 1.0 - sim
    dmin = jnp.min(dist, axis=1, keepdims=True)
    m = 1.0 - dmin
    eq = dist == dmin
    eqf = jnp.where(eq, 1.0, 0.0).astype(jnp.bfloat16)

    parts = lax.dot_general(kmat_ref[...], eqf, (((1,), (1,)), ((), ())),
                            preferred_element_type=jnp.float32)
    hi, lo, cnt = parts[0:1, :], parts[1:2, :], parts[2:3, :]
    ncg, nsqg = parts[3:4, :], parts[4:5, :]
    idxr = lax.round((hi * 32.0 + lo) / cnt,
                     lax.RoundingMethod.TO_NEAREST_EVEN).astype(jnp.int32)
    idx_ref[...] = idxr.reshape(1, 1, BLK)

    ones_row = jnp.ones((1, BLK), jnp.bfloat16)
    acc_ref[...] += lax.dot_general(ones_row, eqf, (((1,), (0,)), ((), ())),
                                    preferred_element_type=jnp.float32)
    nd = lax.dot_general(ncg, dmin, (((1,), (0,)), ((), ())),
                         preferred_element_type=jnp.float32)
    znsq_ref[0, 0] += jnp.sum(zn * zn)
    znsq_ref[0, 1] += jnp.sum(ncg) - jnp.sum(nd)
    znsq_ref[0, 2] += jnp.sum(nsqg)

    @pl.when(pid == GRID - 1)
    def _fin():
        loss_ref[0, 0] = 0.25 * (znsq_ref[0, 0] - 2.0 * znsq_ref[0, 1]
                                 + znsq_ref[0, 2]) \
            / jnp.float32(N_TOKENS * EMBED_DIM)
        p = acc_ref[...] * jnp.float32(1.0 / N_TOKENS)
        ent = -jnp.sum(p * jnp.log(p + 1e-10))
        perp_ref[0, 0] = jnp.exp(ent)


def _tc_call(z, embeddings, interpret=False):
    return pl.pallas_call(
        _tc_body,
        grid=(GRID,),
        in_specs=[
            pl.BlockSpec((BLK, EMBED_DIM), lambda i: (i, 0)),
            pl.BlockSpec((N_CODES, EMBED_DIM), lambda i: (0, 0)),
        ],
        out_specs=[
            pl.BlockSpec((1, 1, BLK), lambda i: (i, 0, 0)),
            pl.BlockSpec(memory_space=pltpu.SMEM),
            pl.BlockSpec(memory_space=pltpu.SMEM),
        ],
        out_shape=[
            jax.ShapeDtypeStruct((GRID, 1, BLK), jnp.int32),
            jax.ShapeDtypeStruct((1, 1), jnp.float32),
            jax.ShapeDtypeStruct((1, 1), jnp.float32),
        ],
        scratch_shapes=[
            pltpu.VMEM((N_CODES, EMBED_DIM), jnp.float32),
            pltpu.VMEM((1, N_CODES), jnp.float32),
            pltpu.SMEM((1, 4), jnp.float32),
            pltpu.VMEM((5, N_CODES), jnp.bfloat16),
        ],
        interpret=interpret,
    )(z, embeddings)


def _sc_gather(embeddings, idx2d):
    info = plsc.get_sparse_core_info()
    nw = info.num_cores * info.num_subcores
    b_per_w = N_TOKENS // nw
    ch = 128
    nch = b_per_w // ch
    mesh = plsc.VectorSubcoreMesh(core_axis_name="c", subcore_axis_name="s")

    nbuf = 3

    @functools.partial(
        pl.kernel, mesh=mesh,
        out_type=jax.ShapeDtypeStruct((N_TOKENS, EMBED_DIM), jnp.float32),
        scratch_types=[
            pltpu.VMEM((nch, ch), jnp.int32),
            pltpu.VMEM((ch, EMBED_DIM), jnp.float32),
            pltpu.VMEM((ch, EMBED_DIM), jnp.float32),
            pltpu.VMEM((ch, EMBED_DIM), jnp.float32),
            pltpu.SemaphoreType.DMA,
            pltpu.SemaphoreType.DMA,
        ],
    )
    def k(table_hbm, idx_hbm, out_hbm, idx_v, rows0, rows1, rows2, gsem, ssem):
        wid = lax.axis_index("s") * info.num_cores + lax.axis_index("c")
        base = wid * b_per_w
        bufs = [rows0, rows1, rows2]
        pltpu.sync_copy(idx_hbm.at[pl.ds(wid * nch, nch)], idx_v)
        gh = [None] * nch
        for j in range(min(2, nch)):
            gh[j] = pltpu.async_copy(table_hbm.at[idx_v.at[j]], bufs[j % nbuf],
                                     gsem)
        stores = [None] * nch
        for j in range(nch):
            gh[j].wait()
            stores[j] = pltpu.async_copy(bufs[j % nbuf],
                                         out_hbm.at[pl.ds(base + j * ch, ch)],
                                         ssem)
            if j + 2 < nch:
                if j >= 1:
                    stores[j - 1].wait()
                gh[j + 2] = pltpu.async_copy(table_hbm.at[idx_v.at[j + 2]],
                                             bufs[(j + 2) % nbuf], gsem)
        stores[nch - 2].wait()
        stores[nch - 1].wait()

    return k(embeddings, idx2d)


def kernel(z, embeddings):
    idx3d, loss, perp = _tc_call(z, embeddings)
    idx = idx3d.reshape(N_TOKENS)
    emb_st = _sc_gather(embeddings, idx.reshape(128, 128))
    return emb_st, idx, loss.reshape(()), perp.reshape(())


# speedup vs baseline: 1.0093x; 1.0093x over previous
import functools

import jax
import jax.numpy as jnp
from jax import lax
from jax.experimental import pallas as pl
from jax.experimental.pallas import tpu as pltpu
from jax.experimental.pallas import tpu_sc as plsc

N_CODES = 1024
EMBED_DIM = 256
N_TOKENS = 16384
BLK = 4096
GRID = N_TOKENS // BLK
EPS = 1e-12


def _tc_body(z_ref, emb_ref, idx_ref, loss_ref, perp_ref,
             embn_ref, nrm_ref, acc_ref, znsq_ref, kmat_ref):
    pid = pl.program_id(0)

    @pl.when(pid == 0)
    def _init():
        e = emb_ref[...]
        ss = jnp.sum(e * e, axis=1, keepdims=True)
        nc = jnp.maximum(jnp.sqrt(ss), EPS)
        embn_ref[...] = e / nc
        nrm_ref[...] = jnp.concatenate([nc, ss], axis=1)
        acc_ref[...] = jnp.zeros((2, N_CODES), jnp.float32)
        znsq_ref[0, 0] = 0.0
        kv = lax.broadcasted_iota(jnp.int32, (3, N_CODES), 1)
        cv = lax.broadcasted_iota(jnp.int32, (3, N_CODES), 0)
        kmat_ref[...] = jnp.where(cv == 0, kv >> 5,
                                  jnp.where(cv == 1, kv & 31, 1)
                                  ).astype(jnp.bfloat16)

    z = z_ref[...]
    zn = z / jnp.maximum(jnp.sqrt(jnp.sum(z * z, axis=1, keepdims=True)), EPS)

    sim = lax.dot_general(zn, embn_ref[...], (((1,), (1,)), ((), ())),
                          preferred_element_type=jnp.float32)

    dist = 1.0 - sim
    dmin = jnp.min(dist, axis=1, keepdims=True)
    m = 1.0 - dmin
    eq = dist == dmin
    eqf = jnp.where(eq, 1.0, 0.0).astype(jnp.bfloat16)

    parts = lax.dot_general(kmat_ref[...], eqf, (((1,), (1,)), ((), ())),
                            preferred_element_type=jnp.float32)
    hi, lo, cnt = parts[0:1, :], parts[1:2, :], parts[2:3, :]
    idxr = lax.round((hi * 32.0 + lo) / cnt,
                     lax.RoundingMethod.TO_NEAREST_EVEN).astype(jnp.int32)
    idx_ref[...] = idxr.reshape(1, 1, BLK)

    lhs = jnp.concatenate([jnp.ones((1, BLK), jnp.float32),
                           m.reshape(1, BLK)], axis=0).astype(jnp.bfloat16)
    acc_ref[...] += lax.dot_general(lhs, eqf, (((1,), (0,)), ((), ())),
                                    preferred_element_type=jnp.float32)
    znsq_ref[0, 0] += jnp.sum(zn * zn)

    @pl.when(pid == GRID - 1)
    def _fin():
        acc = acc_ref[...]
        res = lax.dot_general(acc, nrm_ref[...], (((1,), (0,)), ((), ())),
                              preferred_element_type=jnp.float32,
                              precision=lax.Precision.HIGHEST)
        i0 = lax.broadcasted_iota(jnp.int32, (2, 2), 0)
        i1 = lax.broadcasted_iota(jnp.int32, (2, 2), 1)
        cnt_nsq = jnp.sum(jnp.where((i0 == 0) & (i1 == 1), res, 0.0))
        s_nc = jnp.sum(jnp.where((i0 == 1) & (i1 == 0), res, 0.0))
        loss_ref[0, 0] = 0.25 * (znsq_ref[0, 0] - 2.0 * s_nc + cnt_nsq) \
            / jnp.float32(N_TOKENS * EMBED_DIM)
        p = acc[0:1, :] * jnp.float32(1.0 / N_TOKENS)
        ent = -jnp.sum(p * jnp.log(p + 1e-10))
        perp_ref[0, 0] = jnp.exp(ent)


def _tc_call(z, embeddings, interpret=False):
    return pl.pallas_call(
        _tc_body,
        grid=(GRID,),
        in_specs=[
            pl.BlockSpec((BLK, EMBED_DIM), lambda i: (i, 0)),
            pl.BlockSpec((N_CODES, EMBED_DIM), lambda i: (0, 0)),
        ],
        out_specs=[
            pl.BlockSpec((1, 1, BLK), lambda i: (i, 0, 0)),
            pl.BlockSpec(memory_space=pltpu.SMEM),
            pl.BlockSpec(memory_space=pltpu.SMEM),
        ],
        out_shape=[
            jax.ShapeDtypeStruct((GRID, 1, BLK), jnp.int32),
            jax.ShapeDtypeStruct((1, 1), jnp.float32),
            jax.ShapeDtypeStruct((1, 1), jnp.float32),
        ],
        scratch_shapes=[
            pltpu.VMEM((N_CODES, EMBED_DIM), jnp.float32),
            pltpu.VMEM((N_CODES, 2), jnp.float32),
            pltpu.VMEM((2, N_CODES), jnp.float32),
            pltpu.SMEM((1, 1), jnp.float32),
            pltpu.VMEM((3, N_CODES), jnp.bfloat16),
        ],
        interpret=interpret,
    )(z, embeddings)


def _sc_gather(embeddings, idx2d):
    info = plsc.get_sparse_core_info()
    nw = info.num_cores * info.num_subcores
    b_per_w = N_TOKENS // nw
    ch = 128
    nch = b_per_w // ch
    mesh = plsc.VectorSubcoreMesh(core_axis_name="c", subcore_axis_name="s")

    nbuf = 3

    @functools.partial(
        pl.kernel, mesh=mesh,
        out_type=jax.ShapeDtypeStruct((N_TOKENS, EMBED_DIM), jnp.float32),
        scratch_types=[
            pltpu.VMEM((nch, ch), jnp.int32),
            pltpu.VMEM((ch, EMBED_DIM), jnp.float32),
            pltpu.VMEM((ch, EMBED_DIM), jnp.float32),
            pltpu.VMEM((ch, EMBED_DIM), jnp.float32),
            pltpu.SemaphoreType.DMA,
            pltpu.SemaphoreType.DMA,
        ],
    )
    def k(table_hbm, idx_hbm, out_hbm, idx_v, rows0, rows1, rows2, gsem, ssem):
        wid = lax.axis_index("s") * info.num_cores + lax.axis_index("c")
        base = wid * b_per_w
        bufs = [rows0, rows1, rows2]
        pltpu.sync_copy(idx_hbm.at[pl.ds(wid * nch, nch)], idx_v)
        gh = [None] * nch
        for j in range(min(2, nch)):
            gh[j] = pltpu.async_copy(table_hbm.at[idx_v.at[j]], bufs[j % nbuf],
                                     gsem)
        stores = [None] * nch
        for j in range(nch):
            gh[j].wait()
            stores[j] = pltpu.async_copy(bufs[j % nbuf],
                                         out_hbm.at[pl.ds(base + j * ch, ch)],
                                         ssem)
            if j + 2 < nch:
                if j >= 1:
                    stores[j - 1].wait()
                gh[j + 2] = pltpu.async_copy(table_hbm.at[idx_v.at[j + 2]],
                                             bufs[(j + 2) % nbuf], gsem)
        stores[nch - 2].wait()
        stores[nch - 1].wait()

    return k(embeddings, idx2d)


def kernel(z, embeddings):
    idx3d, loss, perp = _tc_call(z, embeddings)
    idx = idx3d.reshape(N_TOKENS)
    emb_st = _sc_gather(embeddings, idx.reshape(128, 128))
    return emb_st, idx, loss.reshape(()), perp.reshape(())
